# baseline (speedup 1.0000x reference)
import functools

import jax
import jax.numpy as jnp
from jax import lax
from jax.experimental import pallas as pl
from jax.experimental.pallas import tpu as pltpu
from jax.experimental.pallas import tpu_sc as plsc

BATCH = 16384
EMBED_DIM = 64
VOCAB = 100000

_info = plsc.get_sparse_core_info()
_NC, _NS, _NL = _info.num_cores, _info.num_subcores, _info.num_lanes
_NW = _NC * _NS
_ROWS_PER_W = EMBED_DIM // _NW
_CHUNK = 4096
_NCHUNK = BATCH // _CHUNK
_NBUF = 3

_mesh = plsc.VectorSubcoreMesh(core_axis_name="c", subcore_axis_name="s")


@functools.partial(
    pl.kernel,
    mesh=_mesh,
    compiler_params=pltpu.CompilerParams(needs_layout_passes=False),
    out_type=[
        jax.ShapeDtypeStruct((EMBED_DIM, BATCH), jnp.float32),
        jax.ShapeDtypeStruct((EMBED_DIM, BATCH), jnp.float32),
    ],
    scratch_types=[
        pltpu.VMEM((VOCAB,), jnp.float32),
        pltpu.VMEM((BATCH,), jnp.int32),
        pltpu.VMEM((_NBUF * _CHUNK,), jnp.float32),
        pltpu.SemaphoreType.DMA,
        pltpu.SemaphoreType.DMA,
    ],
)
def _lane_gather(sid_hbm, tid_hbm, w1t_hbm, w2t_hbm, o1t_hbm, o2t_hbm,
                 row_v, idx_v, out_v, row_sem, out_sem):
    wid = lax.axis_index("c") * _NS + lax.axis_index("s")

    def do_table(idx_hbm, wt_hbm, ot_hbm):
        pltpu.sync_copy(idx_hbm, idx_v)
        copies = []
        for r in range(_ROWS_PER_W):
            d = wid * _ROWS_PER_W + r
            if r == 0:
                pltpu.async_copy(wt_hbm.at[d], row_v, row_sem).wait()
            for c in range(_NCHUNK):
                buf = (r * _NCHUNK + c) % _NBUF
                if len(copies) >= _NBUF - 1:
                    copies.pop(0).wait()

                @plsc.parallel_loop(0, _CHUNK, _NL)
                def gather_body(j):
                    ids = idx_v[pl.ds(c * _CHUNK + j, _NL)]
                    out_v[pl.ds(buf * _CHUNK + j, _NL)] = plsc.load_gather(
                        row_v, [ids])

                if r + 1 < _ROWS_PER_W and c == _NCHUNK - 1:
                    row_cp = pltpu.async_copy(
                        wt_hbm.at[d + 1], row_v, row_sem)
                copies.append(pltpu.async_copy(
                    out_v.at[pl.ds(buf * _CHUNK, _CHUNK)],
                    ot_hbm.at[d, pl.ds(c * _CHUNK, _CHUNK)],
                    out_sem))
            if r + 1 < _ROWS_PER_W:
                row_cp.wait()
        for cp in copies:
            cp.wait()

    do_table(sid_hbm, w1t_hbm, o1t_hbm)
    do_table(tid_hbm, w2t_hbm, o2t_hbm)


def kernel(speaker_id, target_speaker_id, speaker_embed_weight,
           vocoder_embed_weight):
    o1t, o2t = _lane_gather(speaker_id, target_speaker_id,
                            speaker_embed_weight.T, vocoder_embed_weight.T)
    return (o1t.T, o2t.T)


# speedup vs baseline: 1.3432x; 1.3432x over previous
import functools

import jax
import jax.numpy as jnp
from jax import lax
from jax.experimental import pallas as pl
from jax.experimental.pallas import tpu as pltpu
from jax.experimental.pallas import tpu_sc as plsc

BATCH = 16384
EMBED_DIM = 64
VOCAB = 100000

_info = plsc.get_sparse_core_info()
_NC, _NS, _NL = _info.num_cores, _info.num_subcores, _info.num_lanes
_NW = _NC * _NS
_ROWS_PER_W = EMBED_DIM // _NW
_CHUNK = 4096
_NCHUNK = BATCH // _CHUNK
_NBUF = 3

_mesh = plsc.VectorSubcoreMesh(core_axis_name="c", subcore_axis_name="s")


@functools.partial(
    pl.kernel,
    mesh=_mesh,
    compiler_params=pltpu.CompilerParams(needs_layout_passes=False),
    out_type=[
        jax.ShapeDtypeStruct((EMBED_DIM, BATCH), jnp.float32),
        jax.ShapeDtypeStruct((EMBED_DIM, BATCH), jnp.float32),
    ],
    scratch_types=[
        pltpu.VMEM((VOCAB,), jnp.float32),
        pltpu.VMEM((BATCH,), jnp.int32),
        pltpu.VMEM((_NBUF * _CHUNK,), jnp.float32),
        pltpu.SemaphoreType.DMA,
        pltpu.SemaphoreType.DMA,
    ],
)
def _lane_gather(sid_hbm, tid_hbm, w1t_hbm, w2t_hbm, o1t_hbm, o2t_hbm,
                 row_v, idx_v, out_v, row_sem, out_sem):
    wid = lax.axis_index("c") * _NS + lax.axis_index("s")

    def do_table(idx_hbm, wt_hbm, ot_hbm):
        pltpu.sync_copy(idx_hbm, idx_v)
        copies = []
        for r in range(_ROWS_PER_W):
            d = wid * _ROWS_PER_W + r
            if r == 0:
                pltpu.async_copy(wt_hbm.at[d], row_v, row_sem).wait()
            for c in range(_NCHUNK):
                buf = (r * _NCHUNK + c) % _NBUF
                if len(copies) >= _NBUF - 1:
                    copies.pop(0).wait()

                @plsc.parallel_loop(0, _CHUNK, _NL, unroll=8)
                def gather_body(j):
                    ids = idx_v[pl.ds(c * _CHUNK + j, _NL)]
                    out_v[pl.ds(buf * _CHUNK + j, _NL)] = plsc.load_gather(
                        row_v, [ids])

                if r + 1 < _ROWS_PER_W and c == _NCHUNK - 1:
                    row_cp = pltpu.async_copy(
                        wt_hbm.at[d + 1], row_v, row_sem)
                copies.append(pltpu.async_copy(
                    out_v.at[pl.ds(buf * _CHUNK, _CHUNK)],
                    ot_hbm.at[d, pl.ds(c * _CHUNK, _CHUNK)],
                    out_sem))
            if r + 1 < _ROWS_PER_W:
                row_cp.wait()
        for cp in copies:
            cp.wait()

    do_table(sid_hbm, w1t_hbm, o1t_hbm)
    do_table(tid_hbm, w2t_hbm, o2t_hbm)


def kernel(speaker_id, target_speaker_id, speaker_embed_weight,
           vocoder_embed_weight):
    o1t, o2t = _lane_gather(speaker_id, target_speaker_id,
                            speaker_embed_weight.T, vocoder_embed_weight.T)
    return (o1t.T, o2t.T)


# speedup vs baseline: 1.3580x; 1.0111x over previous
import functools

import jax
import jax.numpy as jnp
from jax import lax
from jax.experimental import pallas as pl
from jax.experimental.pallas import tpu as pltpu
from jax.experimental.pallas import tpu_sc as plsc

BATCH = 16384
EMBED_DIM = 64
VOCAB = 100000

_info = plsc.get_sparse_core_info()
_NC, _NS, _NL = _info.num_cores, _info.num_subcores, _info.num_lanes
_NW = _NC * _NS
_ROWS_PER_W = EMBED_DIM // _NW
_CHUNK = 4096
_NCHUNK = BATCH // _CHUNK
_NBUF = 3

_mesh = plsc.VectorSubcoreMesh(core_axis_name="c", subcore_axis_name="s")


@functools.partial(
    pl.kernel,
    mesh=_mesh,
    compiler_params=pltpu.CompilerParams(needs_layout_passes=False),
    out_type=[
        jax.ShapeDtypeStruct((EMBED_DIM, BATCH), jnp.float32),
        jax.ShapeDtypeStruct((EMBED_DIM, BATCH), jnp.float32),
    ],
    scratch_types=[
        pltpu.VMEM((VOCAB,), jnp.float32),
        pltpu.VMEM((BATCH,), jnp.int32),
        pltpu.VMEM((_NBUF * _CHUNK,), jnp.float32),
        pltpu.SemaphoreType.DMA,
        pltpu.SemaphoreType.DMA,
        pltpu.SemaphoreType.DMA,
    ],
)
def _lane_gather(sid_hbm, tid_hbm, w1t_hbm, w2t_hbm, o1t_hbm, o2t_hbm,
                 row_v, idx_v, out_v, row_sem, out_sem, idx_sem):
    wid = lax.axis_index("c") * _NS + lax.axis_index("s")

    def fire_row(wt_hbm, d):
        return [pltpu.async_copy(wt_hbm.at[d], row_v, row_sem)]

    def fire_idx(idx_hbm):
        return [pltpu.async_copy(idx_hbm, idx_v, idx_sem)]

    tasks = [(w1t_hbm, o1t_hbm, 0), (w1t_hbm, o1t_hbm, 1),
             (w2t_hbm, o2t_hbm, 0), (w2t_hbm, o2t_hbm, 1)]

    pending = fire_idx(sid_hbm) + fire_row(w1t_hbm, wid * _ROWS_PER_W)
    copies = []
    for t, (wt_hbm, ot_hbm, r) in enumerate(tasks):
        d = wid * _ROWS_PER_W + r
        for cp in pending:
            cp.wait()
        pending = []
        for c in range(_NCHUNK):
            buf = (t * _NCHUNK + c) % _NBUF
            if len(copies) >= _NBUF - 1:
                copies.pop(0).wait()

            @plsc.parallel_loop(0, _CHUNK, _NL, unroll=8)
            def gather_body(j):
                ids = idx_v[pl.ds(c * _CHUNK + j, _NL)]
                out_v[pl.ds(buf * _CHUNK + j, _NL)] = plsc.load_gather(
                    row_v, [ids])

            if c == _NCHUNK - 1 and t + 1 < len(tasks):
                nwt, _, nr = tasks[t + 1]
                pending = fire_row(nwt, wid * _ROWS_PER_W + nr)
                if t == 1:
                    pending += fire_idx(tid_hbm)
            copies.append(pltpu.async_copy(
                out_v.at[pl.ds(buf * _CHUNK, _CHUNK)],
                ot_hbm.at[d, pl.ds(c * _CHUNK, _CHUNK)],
                out_sem))
    for cp in copies:
        cp.wait()


def kernel(speaker_id, target_speaker_id, speaker_embed_weight,
           vocoder_embed_weight):
    o1t, o2t = _lane_gather(speaker_id, target_speaker_id,
                            speaker_embed_weight.T, vocoder_embed_weight.T)
    return (o1t.T, o2t.T)
